# baseline (speedup 1.0000x reference)
import functools

import jax
import jax.numpy as jnp
from jax import lax
from jax.experimental import pallas as pl
from jax.experimental.pallas import tpu as pltpu
from jax.experimental.pallas import tpu_sc as plsc

N_NODES = 10000
N_EDGES = 320000
D_FEAT = 128

_INFO = plsc.get_sparse_core_info()
_NC = _INFO.num_cores
_NS = _INFO.num_subcores
_NW = _NC * _NS
_L = _INFO.num_lanes

_C = 64
_NCHUNKS = N_EDGES // _C
_ROUNDS = -(-_NCHUNKS // _NW)


def _make_kernel():
    mesh = plsc.VectorSubcoreMesh(core_axis_name="c", subcore_axis_name="s")

    @functools.partial(
        pl.kernel,
        mesh=mesh,
        out_type=jax.ShapeDtypeStruct((N_EDGES,), jnp.float32),
        scratch_types=[
            pltpu.VMEM((2, _C), jnp.int32),
            pltpu.VMEM((2, _C), jnp.int32),
            pltpu.VMEM((_C, D_FEAT), jnp.float32),
            pltpu.VMEM((_C, D_FEAT), jnp.float32),
            pltpu.VMEM((_C, D_FEAT), jnp.float32),
            pltpu.VMEM((_C, D_FEAT), jnp.float32),
            pltpu.VMEM((_C,), jnp.float32),
            pltpu.VMEM((_C,), jnp.float32),
            pltpu.VMEM((_L, 17), jnp.float32),
            pltpu.VMEM((_L, 17), jnp.float32),
            pltpu.VMEM_SHARED((N_NODES, D_FEAT), jnp.float32),
            pltpu.SemaphoreType.DMA,
            pltpu.SemaphoreType.DMA,
            pltpu.SemaphoreType.DMA,
            pltpu.SemaphoreType.DMA,
            pltpu.SemaphoreType.DMA,
            pltpu.SemaphoreType.DMA,
            pltpu.SemaphoreType.DMA,
            pltpu.SemaphoreType.DMA,
        ],
        compiler_params=pltpu.CompilerParams(needs_layout_passes=False),
    )
    def dot_scores(idx_hbm, h_hbm, out_hbm,
                   i0, i1, u0, u1, v0, v1, s0, s1, pmat0, pmat1, h_sp,
                   su0, su1, sv0, sv1, si0, si1, so0, so1):
        wid = lax.axis_index("s") * _NC + lax.axis_index("c")
        sid = lax.axis_index("s")
        lanes = lax.iota(jnp.int32, _L)

        idx_slot = [i0, i1]
        u_slot = [u0, u1]
        v_slot = [v0, v1]
        sc_slot = [s0, s1]
        su_slot = [su0, su1]
        sv_slot = [sv0, sv1]
        si_slot = [si0, si1]
        so_slot = [so0, so1]

        rows_per_tile = 624
        stage0 = sid * rows_per_tile
        pltpu.sync_copy(h_hbm.at[pl.ds(stage0, rows_per_tile)],
                        h_sp.at[pl.ds(stage0, rows_per_tile)])

        @pl.when(sid == 0)
        def _():
            tail = N_NODES - _NS * rows_per_tile
            pltpu.sync_copy(h_hbm.at[pl.ds(_NS * rows_per_tile, tail)],
                            h_sp.at[pl.ds(_NS * rows_per_tile, tail)])

        plsc.subcore_barrier()

        def cid_of(t):
            return t * _NW + wid

        def idx_copy(t, slot):
            @pl.when(cid_of(t) < _NCHUNKS)
            def _():
                pltpu.async_copy(idx_hbm.at[cid_of(t)], idx_slot[slot],
                                 si_slot[slot])

        def idx_wait(t, slot):
            @pl.when(cid_of(t) < _NCHUNKS)
            def _():
                pltpu.make_async_copy(idx_hbm.at[cid_of(t)], idx_slot[slot],
                                      si_slot[slot]).wait()

        def gathers_issue(t, slot):
            @pl.when(cid_of(t) < _NCHUNKS)
            def _():
                pltpu.async_copy(h_sp.at[idx_slot[slot].at[0]],
                                 u_slot[slot], su_slot[slot])
                pltpu.async_copy(h_sp.at[idx_slot[slot].at[1]],
                                 v_slot[slot], sv_slot[slot])

        def gathers_wait(t, slot):
            @pl.when(cid_of(t) < _NCHUNKS)
            def _():
                pltpu.make_async_copy(h_sp.at[idx_slot[slot].at[0]],
                                      u_slot[slot], su_slot[slot]).wait()
                pltpu.make_async_copy(h_sp.at[idx_slot[slot].at[1]],
                                      v_slot[slot], sv_slot[slot]).wait()

        def out_drain(slot):
            pltpu.make_async_copy(sc_slot[slot], out_hbm.at[pl.ds(0, _C)],
                                  so_slot[slot]).wait()

        def compute(t, slot):
            cid = cid_of(t)

            @pl.when(cid < _NCHUNKS)
            def _():
                u_rows = u_slot[slot]
                v_rows = v_slot[slot]
                scores = sc_slot[slot]

                @pl.when(t >= 2)
                def _():
                    out_drain(slot)

                def edge_loads(e):
                    us = [u_rows[e, pl.ds(k * _L, _L)]
                          for k in range(D_FEAT // _L)]
                    vs = [v_rows[e, pl.ds(k * _L, _L)]
                          for k in range(D_FEAT // _L)]
                    return us, vs

                def edge_arith(pmat, j, us, vs):
                    parts = [us[k] * vs[k] for k in range(D_FEAT // _L)]
                    while len(parts) > 1:
                        parts = [parts[i] + parts[i + 1]
                                 for i in range(0, len(parts), 2)]
                    pmat[j, pl.ds(0, _L)] = parts[0]

                def group_transpose(pmat, g):
                    rows = g * _L + lanes
                    gath = [plsc.load_gather(
                                pmat, [lanes, jnp.full((_L,), k, jnp.int32)])
                            for k in range(_L)]
                    while len(gath) > 1:
                        gath = [gath[i] + gath[i + 1]
                                for i in range(0, len(gath), 2)]
                    plsc.store_scatter(scores, [rows], gath[0])

                pmats = [pmat0, pmat1]

                def pair_of_groups(gg, carry):
                    pending = None
                    for b in range(2):
                        g = gg * 2 + b
                        pmat = pmats[b]
                        prev = edge_loads(g * _L)
                        for j in range(_L):
                            cur = (edge_loads(g * _L + j + 1)
                                   if j + 1 < _L else None)
                            edge_arith(pmat, j, *prev)
                            if j == 1 and pending is not None:
                                group_transpose(*pending)
                                pending = None
                            prev = cur
                        pending = (pmat, g)
                    group_transpose(*pending)
                    return carry

                lax.fori_loop(0, _C // _L // 2, pair_of_groups, 0)
                pltpu.async_copy(scores, out_hbm.at[pl.ds(cid * _C, _C)],
                                 so_slot[slot])

        def step(t, slot):
            other = 1 - slot
            idx_wait(t + 1, other)
            gathers_issue(t + 1, other)
            gathers_wait(t, slot)
            idx_copy(t + 2, slot)
            compute(t, slot)

        idx_copy(jnp.int32(0), 0)
        idx_wait(jnp.int32(0), 0)
        gathers_issue(jnp.int32(0), 0)
        idx_copy(jnp.int32(1), 1)

        def pair_body(p, carry):
            t0 = p * 2
            step(t0, 0)
            step(t0 + 1, 1)
            return carry

        lax.fori_loop(0, (_ROUNDS + 1) // 2, pair_body, 0)

        n_valid = (_NCHUNKS - wid + _NW - 1) // _NW

        for s in (0, 1):
            @pl.when(n_valid > s)
            def _(s=s):
                out_drain(s)

    return dot_scores


_dot_scores = _make_kernel()


def kernel(h, edge_index):
    idx = edge_index.astype(jnp.int32).reshape(2, _NCHUNKS, _C)
    idx_packed = idx.transpose(1, 0, 2)
    return _dot_scores(idx_packed, h)
